# Initial kernel scaffold; baseline (speedup 1.0000x reference)
import jax
import jax.numpy as jnp
from jax.experimental import pallas as pl


def kernel(f0, f1, f2, f3, f4, f5, f6, f7, f8, f9, f10, f11, f12, f13, f14, f15, f16, f17, f18, f19, f20, f21, f22, f23, f24, f25, T0, T1, T2, T3, T4, T5, T6, T7, T8, T9, T10, T11, T12, T13, T14, T15, T16, T17, T18, T19, T20, T21, T22, T23, T24, T25):
    raise NotImplementedError("write your pallas kernel here")


# speedup vs baseline: 4.9132x; 4.9132x over previous
import functools

import jax
import jax.numpy as jnp
from jax import lax
from jax.experimental import pallas as pl
from jax.experimental.pallas import tpu as pltpu
from jax.experimental.pallas import tpu_sc as plsc

_NUM_FIELDS = 26
_BATCH = 16384
_ROWS = 1001
_EMB = 128

_NC, _NS = 2, 16
_NW = _NC * _NS
_BPW = _BATCH // _NW
_CHUNK = 128
_NCHUNK = _BPW // _CHUNK

_mesh = plsc.VectorSubcoreMesh(core_axis_name="c", subcore_axis_name="s")


@functools.partial(
    pl.kernel,
    mesh=_mesh,
    out_type=jax.ShapeDtypeStruct((_BATCH, _NUM_FIELDS * _EMB), jnp.float32),
    scratch_types=[
        pltpu.VMEM((_NUM_FIELDS, _NCHUNK, _CHUNK), jnp.int32),
        pltpu.VMEM((_CHUNK, _EMB), jnp.float32),
        pltpu.SemaphoreType.DMA,
    ],
)
def _sc_gather(tables, idx, out, idx_v, rows_v, sem):
    wid = lax.axis_index("s") * _NC + lax.axis_index("c")
    base = wid * _BPW
    pltpu.sync_copy(idx.at[:, pl.ds(wid * _NCHUNK, _NCHUNK), :], idx_v)

    def step(t, carry):
        i = t // _NCHUNK
        j = t % _NCHUNK
        pltpu.async_copy(tables.at[idx_v.at[i, j]], rows_v, sem).wait()
        pltpu.sync_copy(
            rows_v,
            out.at[pl.ds(base + j * _CHUNK, _CHUNK), pl.ds(i * _EMB, _EMB)],
        )
        return carry

    lax.fori_loop(0, _NUM_FIELDS * _NCHUNK, step, 0)


def kernel(f0, f1, f2, f3, f4, f5, f6, f7, f8, f9, f10, f11, f12, f13,
           f14, f15, f16, f17, f18, f19, f20, f21, f22, f23, f24, f25,
           T0, T1, T2, T3, T4, T5, T6, T7, T8, T9, T10, T11, T12, T13,
           T14, T15, T16, T17, T18, T19, T20, T21, T22, T23, T24, T25):
    fields = [f0, f1, f2, f3, f4, f5, f6, f7, f8, f9, f10, f11, f12, f13,
              f14, f15, f16, f17, f18, f19, f20, f21, f22, f23, f24, f25]
    tables = [T0, T1, T2, T3, T4, T5, T6, T7, T8, T9, T10, T11, T12, T13,
              T14, T15, T16, T17, T18, T19, T20, T21, T22, T23, T24, T25]
    flat_tables = jnp.concatenate(tables, axis=0)
    idx = jnp.stack(
        [f.astype(jnp.int32) + i * _ROWS for i, f in enumerate(fields)]
    ).reshape(_NUM_FIELDS, _BATCH // _CHUNK, _CHUNK)
    return _sc_gather(flat_tables, idx)


# speedup vs baseline: 6.4076x; 1.3042x over previous
import functools

import jax
import jax.numpy as jnp
from jax import lax
from jax.experimental import pallas as pl
from jax.experimental.pallas import tpu as pltpu
from jax.experimental.pallas import tpu_sc as plsc

_NUM_FIELDS = 26
_BATCH = 16384
_ROWS = 1001
_EMB = 128

_NC, _NS = 2, 16
_NW = _NC * _NS
_BPW = _BATCH // _NW
_CHUNK = 128
_NCHUNK = _BPW // _CHUNK

_mesh = plsc.VectorSubcoreMesh(core_axis_name="c", subcore_axis_name="s")


_NBUF = 4
_NSTEP = _NUM_FIELDS * _NCHUNK


@functools.partial(
    pl.kernel,
    mesh=_mesh,
    out_type=jax.ShapeDtypeStruct((_BATCH, _NUM_FIELDS * _EMB), jnp.float32),
    scratch_types=[
        pltpu.VMEM((_NUM_FIELDS, _NCHUNK, _CHUNK), jnp.int32),
        [pltpu.VMEM((_CHUNK, _EMB), jnp.float32) for _ in range(_NBUF)],
        [pltpu.SemaphoreType.DMA for _ in range(_NBUF)],
        [pltpu.SemaphoreType.DMA for _ in range(_NBUF)],
    ],
)
def _sc_gather(tables, idx, out, idx_v, bufs, gsems, ssems):
    wid = lax.axis_index("s") * _NC + lax.axis_index("c")
    base = wid * _BPW
    pltpu.sync_copy(idx.at[:, pl.ds(wid * _NCHUNK, _NCHUNK), :], idx_v)

    def start_gather(t, b):
        i = t // _NCHUNK
        j = t % _NCHUNK
        pltpu.async_copy(tables.at[idx_v.at[i, j]], bufs[b], gsems[b])

    def start_scatter(t, b):
        i = t // _NCHUNK
        j = t % _NCHUNK
        pltpu.async_copy(
            bufs[b],
            out.at[pl.ds(base + j * _CHUNK, _CHUNK), pl.ds(i * _EMB, _EMB)],
            ssems[b],
        )

    def wait_gather(b):
        pltpu.make_async_copy(tables.at[idx_v.at[0, 0]], bufs[b], gsems[b]).wait()

    def wait_scatter(b):
        pltpu.make_async_copy(
            bufs[b], out.at[pl.ds(base, _CHUNK), pl.ds(0, _EMB)], ssems[b]
        ).wait()

    for b in range(_NBUF):
        start_gather(b, b)

    def ring_step(t, b, refill):
        wait_gather(b)
        start_scatter(t, b)
        if refill:
            wait_scatter(b)
            start_gather(t + _NBUF, b)

    def outer(p, carry):
        for b in range(_NBUF):
            ring_step(p * _NBUF + b, b, refill=True)
        return carry

    lax.fori_loop(0, _NSTEP // _NBUF - 1, outer, 0)
    for b in range(_NBUF):
        ring_step(_NSTEP - _NBUF + b, b, refill=False)
    for b in range(_NBUF):
        wait_scatter(b)


def kernel(f0, f1, f2, f3, f4, f5, f6, f7, f8, f9, f10, f11, f12, f13,
           f14, f15, f16, f17, f18, f19, f20, f21, f22, f23, f24, f25,
           T0, T1, T2, T3, T4, T5, T6, T7, T8, T9, T10, T11, T12, T13,
           T14, T15, T16, T17, T18, T19, T20, T21, T22, T23, T24, T25):
    fields = [f0, f1, f2, f3, f4, f5, f6, f7, f8, f9, f10, f11, f12, f13,
              f14, f15, f16, f17, f18, f19, f20, f21, f22, f23, f24, f25]
    tables = [T0, T1, T2, T3, T4, T5, T6, T7, T8, T9, T10, T11, T12, T13,
              T14, T15, T16, T17, T18, T19, T20, T21, T22, T23, T24, T25]
    flat_tables = jnp.concatenate(tables, axis=0)
    idx = jnp.stack(
        [f.astype(jnp.int32) + i * _ROWS for i, f in enumerate(fields)]
    ).reshape(_NUM_FIELDS, _BATCH // _CHUNK, _CHUNK)
    return _sc_gather(flat_tables, idx)


# speedup vs baseline: 8.8880x; 1.3871x over previous
import functools

import jax
import jax.numpy as jnp
from jax import lax
from jax.experimental import pallas as pl
from jax.experimental.pallas import tpu as pltpu
from jax.experimental.pallas import tpu_sc as plsc

_NUM_FIELDS = 26
_BATCH = 16384
_EMB = 128

_NC, _NS = 2, 16
_NW = _NC * _NS
_BPW = _BATCH // _NW
_CHUNK = 128
_NCHUNK = _BPW // _CHUNK
_NBUF = 4
_NSTEP = _NUM_FIELDS * _NCHUNK

_mesh = plsc.VectorSubcoreMesh(core_axis_name="c", subcore_axis_name="s")


@functools.partial(
    pl.kernel,
    mesh=_mesh,
    out_type=jax.ShapeDtypeStruct((_BATCH, _NUM_FIELDS * _EMB), jnp.float32),
    scratch_types=[
        pltpu.VMEM((_NUM_FIELDS, _NCHUNK, _CHUNK), jnp.int32),
        [pltpu.VMEM((_CHUNK, _EMB), jnp.float32) for _ in range(_NBUF)],
        pltpu.SemaphoreType.DMA,
        [pltpu.SemaphoreType.DMA for _ in range(_NBUF)],
        [pltpu.SemaphoreType.DMA for _ in range(_NBUF)],
    ],
)
def _sc_gather(*refs):
    tables = refs[:_NUM_FIELDS]
    fields = refs[_NUM_FIELDS:2 * _NUM_FIELDS]
    out, idx_v, bufs, isem, gsems, ssems = refs[2 * _NUM_FIELDS:]
    wid = lax.axis_index("s") * _NC + lax.axis_index("c")
    base = wid * _BPW

    for i in range(_NUM_FIELDS):
        pltpu.async_copy(
            fields[i].at[pl.ds(wid * _NCHUNK, _NCHUNK), :], idx_v.at[i], isem)
    for i in range(_NUM_FIELDS):
        pltpu.make_async_copy(
            fields[i].at[pl.ds(wid * _NCHUNK, _NCHUNK), :], idx_v.at[i], isem
        ).wait()

    def start_gather(t, b):
        i, j = divmod(t, _NCHUNK)
        pltpu.async_copy(
            tables[i].at[idx_v.at[i, j]],
            bufs[b], gsems[b],
        )

    def start_scatter(t, b):
        i, j = divmod(t, _NCHUNK)
        pltpu.async_copy(
            bufs[b],
            out.at[pl.ds(base + j * _CHUNK, _CHUNK),
                   pl.ds(i * _EMB, _EMB)],
            ssems[b],
        )

    def wait_gather(b):
        pltpu.make_async_copy(
            tables[0].at[idx_v.at[0, 0]], bufs[b], gsems[b]
        ).wait()

    def wait_scatter(b):
        pltpu.make_async_copy(
            bufs[b], out.at[pl.ds(base, _CHUNK), pl.ds(0, _EMB)], ssems[b]
        ).wait()

    for b in range(_NBUF):
        start_gather(b, b)
    for t in range(_NSTEP):
        b = t % _NBUF
        wait_gather(b)
        start_scatter(t, b)
        if 1 <= t and t - 1 + _NBUF < _NSTEP:
            pb = (t - 1) % _NBUF
            wait_scatter(pb)
            start_gather(t - 1 + _NBUF, pb)
    for b in range(_NBUF):
        wait_scatter(b)


def kernel(f0, f1, f2, f3, f4, f5, f6, f7, f8, f9, f10, f11, f12, f13,
           f14, f15, f16, f17, f18, f19, f20, f21, f22, f23, f24, f25,
           T0, T1, T2, T3, T4, T5, T6, T7, T8, T9, T10, T11, T12, T13,
           T14, T15, T16, T17, T18, T19, T20, T21, T22, T23, T24, T25):
    fields = [f0, f1, f2, f3, f4, f5, f6, f7, f8, f9, f10, f11, f12, f13,
              f14, f15, f16, f17, f18, f19, f20, f21, f22, f23, f24, f25]
    tables = [T0, T1, T2, T3, T4, T5, T6, T7, T8, T9, T10, T11, T12, T13,
              T14, T15, T16, T17, T18, T19, T20, T21, T22, T23, T24, T25]
    return _sc_gather(
        *tables,
        *[f.astype(jnp.int32).reshape(_BATCH // _CHUNK, _CHUNK) for f in fields])


# speedup vs baseline: 8.9493x; 1.0069x over previous
import functools

import jax
import jax.numpy as jnp
from jax import lax
from jax.experimental import pallas as pl
from jax.experimental.pallas import tpu as pltpu
from jax.experimental.pallas import tpu_sc as plsc

_NUM_FIELDS = 26
_BATCH = 16384
_EMB = 128

_NC, _NS = 2, 16
_NW = _NC * _NS
_BPW = _BATCH // _NW
_CHUNK = 128
_NCHUNK = _BPW // _CHUNK
_NBUF = 6
_LAG = 2
_NSTEP = _NUM_FIELDS * _NCHUNK

_mesh = plsc.VectorSubcoreMesh(core_axis_name="c", subcore_axis_name="s")


@functools.partial(
    pl.kernel,
    mesh=_mesh,
    out_type=jax.ShapeDtypeStruct((_BATCH, _NUM_FIELDS * _EMB), jnp.float32),
    scratch_types=[
        pltpu.VMEM((_NUM_FIELDS, _NCHUNK, _CHUNK), jnp.int32),
        [pltpu.VMEM((_CHUNK, _EMB), jnp.float32) for _ in range(_NBUF)],
        pltpu.SemaphoreType.DMA,
        [pltpu.SemaphoreType.DMA for _ in range(_NBUF)],
        [pltpu.SemaphoreType.DMA for _ in range(_NBUF)],
    ],
)
def _sc_gather(*refs):
    tables = refs[:_NUM_FIELDS]
    fields = refs[_NUM_FIELDS:2 * _NUM_FIELDS]
    out, idx_v, bufs, isem, gsems, ssems = refs[2 * _NUM_FIELDS:]
    wid = lax.axis_index("s") * _NC + lax.axis_index("c")
    base = wid * _BPW

    for i in range(_NUM_FIELDS):
        pltpu.async_copy(
            fields[i].at[pl.ds(wid * _NCHUNK, _NCHUNK), :], idx_v.at[i], isem)
    for i in range(_NUM_FIELDS):
        pltpu.make_async_copy(
            fields[i].at[pl.ds(wid * _NCHUNK, _NCHUNK), :], idx_v.at[i], isem
        ).wait()

    def start_gather(t, b):
        i, j = divmod(t, _NCHUNK)
        pltpu.async_copy(
            tables[i].at[idx_v.at[i, j]],
            bufs[b], gsems[b],
        )

    def start_scatter(t, b):
        i, j = divmod(t, _NCHUNK)
        pltpu.async_copy(
            bufs[b],
            out.at[pl.ds(base + j * _CHUNK, _CHUNK),
                   pl.ds(i * _EMB, _EMB)],
            ssems[b],
        )

    def wait_gather(b):
        pltpu.make_async_copy(
            tables[0].at[idx_v.at[0, 0]], bufs[b], gsems[b]
        ).wait()

    def wait_scatter(b):
        pltpu.make_async_copy(
            bufs[b], out.at[pl.ds(base, _CHUNK), pl.ds(0, _EMB)], ssems[b]
        ).wait()

    for b in range(_NBUF):
        start_gather(b, b)
    for t in range(_NSTEP):
        b = t % _NBUF
        wait_gather(b)
        start_scatter(t, b)
        if _LAG <= t and t - _LAG + _NBUF < _NSTEP:
            pb = (t - _LAG) % _NBUF
            wait_scatter(pb)
            start_gather(t - _LAG + _NBUF, pb)
    for b in range(_NBUF):
        wait_scatter(b)


def kernel(f0, f1, f2, f3, f4, f5, f6, f7, f8, f9, f10, f11, f12, f13,
           f14, f15, f16, f17, f18, f19, f20, f21, f22, f23, f24, f25,
           T0, T1, T2, T3, T4, T5, T6, T7, T8, T9, T10, T11, T12, T13,
           T14, T15, T16, T17, T18, T19, T20, T21, T22, T23, T24, T25):
    fields = [f0, f1, f2, f3, f4, f5, f6, f7, f8, f9, f10, f11, f12, f13,
              f14, f15, f16, f17, f18, f19, f20, f21, f22, f23, f24, f25]
    tables = [T0, T1, T2, T3, T4, T5, T6, T7, T8, T9, T10, T11, T12, T13,
              T14, T15, T16, T17, T18, T19, T20, T21, T22, T23, T24, T25]
    return _sc_gather(
        *tables,
        *[f.astype(jnp.int32).reshape(_BATCH // _CHUNK, _CHUNK) for f in fields])
